# baseline (speedup 1.0000x reference)
import functools

import jax
import jax.numpy as jnp
from jax import lax
from jax.experimental import pallas as pl
from jax.experimental.pallas import tpu as pltpu
from jax.experimental.pallas import tpu_sc as plsc

B = 4096
L = 200
DIM = 64
WPAD = 128
NPE = 256
NW = 32
RPW = B // NW
KV = RPW // 16


def _sc_embed(ids_hbm, w_hbm, pe_hbm, emb_hbm, word_hbm, pos_hbm,
              pe_v, ids_v, nz_v, wrow0, wrow1,
              word_t0, word_t1, pos_t0, pos_t1, emb_t0, emb_t1,
              sem_g0, sem_g1, sem_o0, sem_o1):
    wid = lax.axis_index("s") * 2 + lax.axis_index("c")
    base = wid * RPW

    wrow = (wrow0, wrow1)
    word_t = (word_t0, word_t1)
    pos_t = (pos_t0, pos_t1)
    emb_t = (emb_t0, emb_t1)
    sem_g = (sem_g0, sem_g1)
    sem_o = (sem_o0, sem_o1)
    outs = (word_hbm, pos_hbm, emb_hbm)

    pltpu.sync_copy(pe_hbm.at[:, pl.ds(0, NPE)], pe_v)
    pltpu.sync_copy(ids_hbm.at[:, pl.ds(base, RPW)], ids_v)

    lane = lax.iota(jnp.int32, 16)
    perm = {s: lane ^ s for s in (1, 2, 4, 8)}
    sel = {s: (lane & s) == 0 for s in (1, 2, 4, 8)}

    for k in range(KV):
        def cnt(l, acc):
            v = ids_v[l, pl.ds(k * 16, 16)]
            return acc + jnp.where(v != 0, 1, 0).astype(jnp.int32)
        nz_v[pl.ds(k * 16, 16)] = lax.fori_loop(0, L, cnt, jnp.zeros((16,), jnp.int32))

    def start_gather(l, p):
        pltpu.async_copy(w_hbm.at[ids_v.at[l]], wrow[p], sem_g[p])

    def drain_gather(l, p):
        pltpu.make_async_copy(w_hbm.at[ids_v.at[l]], wrow[p], sem_g[p]).wait()

    def bufs(p):
        return (word_t[p], pos_t[p], emb_t[p])

    def start_outs(l, p):
        for o, t in zip(outs, bufs(p)):
            pltpu.async_copy(t, o.at[pl.ds(l * DIM, DIM), pl.ds(base, RPW)],
                             sem_o[p])

    def drain_outs(l, p):
        for o, t in zip(outs, bufs(p)):
            pltpu.make_async_copy(
                t, o.at[pl.ds(l * DIM, DIM), pl.ds(base, RPW)], sem_o[p]).wait()

    def transpose16(k, d0, p):
        v = [wrow[p][k * 16 + r, pl.ds(d0, 16)] for r in range(16)]
        for s in (1, 2, 4, 8):
            nv = list(v)
            for r in range(16):
                if r & s:
                    continue
                a, b = v[r], v[r ^ s]
                pa = jnp.take(a, perm[s], mode="wrap")
                pb = jnp.take(b, perm[s], mode="wrap")
                nv[r] = jnp.where(sel[s], a, pb)
                nv[r ^ s] = jnp.where(sel[s], pa, b)
            v = nv
        return v

    def compute(l, p):
        def kbody(k, carry):
            nz16 = nz_v[pl.ds(k * 16, 16)]
            pcol = jnp.where((l + 1) <= nz16, l + 1, 0).astype(jnp.int32)

            for db in range(DIM // 16):
                d0 = db * 16
                v = transpose16(k, d0, p)
                for c in range(16):
                    d = d0 + c
                    wvec = v[c]
                    pvec = plsc.load_gather(
                        pe_v, [jnp.full((16,), d, jnp.int32), pcol])
                    word_t[p][d, pl.ds(k * 16, 16)] = wvec
                    pos_t[p][d, pl.ds(k * 16, 16)] = pvec
                    emb_t[p][d, pl.ds(k * 16, 16)] = wvec + pvec
            return carry

        lax.fori_loop(0, KV, kbody, 0)

    start_gather(0, 0)

    def step(lc, carry):
        for p in range(2):
            l = lc * 2 + p
            drain_gather(l, p)

            @pl.when(l + 1 < L)
            def _():
                start_gather(l + 1, p ^ 1)

            @pl.when(l >= 2)
            def _():
                drain_outs(l, p)

            compute(l, p)
            start_outs(l, p)
        return carry

    lax.fori_loop(0, L // 2, step, 0)
    drain_outs(L - 2, 0)
    drain_outs(L - 1, 1)


VBLK = 512


def _tc_pad_body(wt_ref, out_ref):
    out_ref[:, 0:DIM] = wt_ref[...].T
    out_ref[:, DIM:WPAD] = jnp.zeros((VBLK, WPAD - DIM), jnp.float32)


def _pad_table(wt):
    v = wt.shape[1]
    grid = (v + VBLK - 1) // VBLK
    return pl.pallas_call(
        _tc_pad_body,
        grid=(grid,),
        in_specs=[pl.BlockSpec((DIM, VBLK), lambda i: (0, i))],
        out_specs=pl.BlockSpec((VBLK, WPAD), lambda i: (i, 0)),
        out_shape=jax.ShapeDtypeStruct((v, WPAD), jnp.float32),
    )(wt)


def kernel(input_ids, W, pe):
    mesh = plsc.VectorSubcoreMesh(core_axis_name="c", subcore_axis_name="s")
    out = jax.ShapeDtypeStruct((L * DIM, B), jnp.float32)
    otile = pltpu.VMEM((DIM, RPW), jnp.float32)
    f = functools.partial(
        pl.kernel,
        mesh=mesh,
        out_type=(out, out, out),
        compiler_params=pltpu.CompilerParams(needs_layout_passes=False),
        scratch_types=[
            pltpu.VMEM((DIM, NPE), jnp.float32),
            pltpu.VMEM((L, RPW), jnp.int32),
            pltpu.VMEM((RPW,), jnp.int32),
            pltpu.VMEM((RPW, WPAD), jnp.float32),
            pltpu.VMEM((RPW, WPAD), jnp.float32),
            otile, otile, otile, otile, otile, otile,
            pltpu.SemaphoreType.DMA,
            pltpu.SemaphoreType.DMA,
            pltpu.SemaphoreType.DMA,
            pltpu.SemaphoreType.DMA,
        ],
    )(_sc_embed)
    w_pad = _pad_table(W.T)
    emb, word, pos = f(input_ids.T, w_pad, pe.T)

    def untrans(o):
        return o.reshape(L, DIM, B).transpose(2, 0, 1)

    return untrans(emb), untrans(word), untrans(pos)


# speedup vs baseline: 1.9351x; 1.9351x over previous
import functools

import jax
import jax.numpy as jnp
from jax import lax
from jax.experimental import pallas as pl
from jax.experimental.pallas import tpu as pltpu
from jax.experimental.pallas import tpu_sc as plsc

B = 4096
L = 200
DIM = 64
WPAD = 128
NPE = 256
NW = 32
RPW = B // NW
KV = RPW // 16


def _sc_embed(ids_hbm, w_hbm, pe_hbm, emb_hbm, word_hbm, pos_hbm,
              pe_v, ids_v, nz_v, wrow0, wrow1,
              word_t0, word_t1, pos_t0, pos_t1, emb_t0, emb_t1,
              sem_g0, sem_g1, sem_o0, sem_o1):
    wid = lax.axis_index("s") * 2 + lax.axis_index("c")
    base = wid * RPW

    wrow = (wrow0, wrow1)
    word_t = (word_t0, word_t1)
    pos_t = (pos_t0, pos_t1)
    emb_t = (emb_t0, emb_t1)
    sem_g = (sem_g0, sem_g1)
    sem_o = (sem_o0, sem_o1)
    outs = (word_hbm, pos_hbm, emb_hbm)

    pltpu.sync_copy(pe_hbm.at[:, pl.ds(0, NPE)], pe_v)
    pltpu.sync_copy(ids_hbm.at[:, pl.ds(base, RPW)], ids_v)

    lane = lax.iota(jnp.int32, 16)
    perm = {s: lane ^ s for s in (1, 2, 4, 8)}
    sel = {s: (lane & s) == 0 for s in (1, 2, 4, 8)}

    for k in range(KV):
        def cnt(l, acc):
            v = ids_v[l, pl.ds(k * 16, 16)]
            return acc + jnp.where(v != 0, 1, 0).astype(jnp.int32)
        nz_v[pl.ds(k * 16, 16)] = lax.fori_loop(0, L, cnt, jnp.zeros((16,), jnp.int32))

    def start_gather(l, p):
        pltpu.async_copy(w_hbm.at[ids_v.at[l]], wrow[p], sem_g[p])

    def drain_gather(l, p):
        pltpu.make_async_copy(w_hbm.at[ids_v.at[l]], wrow[p], sem_g[p]).wait()

    def bufs(p):
        return (word_t[p], pos_t[p], emb_t[p])

    def start_outs(l, p):
        for o, t in zip(outs, bufs(p)):
            pltpu.async_copy(t, o.at[pl.ds(l * DIM, DIM), pl.ds(base, RPW)],
                             sem_o[p])

    def drain_outs(l, p):
        for o, t in zip(outs, bufs(p)):
            pltpu.make_async_copy(
                t, o.at[pl.ds(l * DIM, DIM), pl.ds(base, RPW)], sem_o[p]).wait()

    def transpose16(k, d0, p):
        v = [wrow[p][k * 16 + r, pl.ds(d0, 16)] for r in range(16)]
        for s in (1, 2, 4, 8):
            nv = list(v)
            for r in range(16):
                if r & s:
                    continue
                a, b = v[r], v[r ^ s]
                pa = jnp.take(a, perm[s], mode="wrap")
                pb = jnp.take(b, perm[s], mode="wrap")
                nv[r] = jnp.where(sel[s], a, pb)
                nv[r ^ s] = jnp.where(sel[s], pa, b)
            v = nv
        return v

    def compute(l, p):
        def kbody(k, carry):
            nz16 = nz_v[pl.ds(k * 16, 16)]
            pcol = jnp.where((l + 1) <= nz16, l + 1, 0).astype(jnp.int32)

            for db in range(DIM // 16):
                d0 = db * 16
                v = transpose16(k, d0, p)
                for c in range(16):
                    d = d0 + c
                    wvec = v[c]
                    pvec = plsc.load_gather(
                        pe_v, [jnp.full((16,), d, jnp.int32), pcol])
                    word_t[p][d, pl.ds(k * 16, 16)] = wvec
                    pos_t[p][d, pl.ds(k * 16, 16)] = pvec
                    emb_t[p][d, pl.ds(k * 16, 16)] = wvec + pvec
            return carry

        lax.fori_loop(0, KV, kbody, 0)

    start_gather(0, 0)

    def step(lc, carry):
        for p in range(2):
            l = lc * 2 + p
            drain_gather(l, p)

            @pl.when(l + 1 < L)
            def _():
                start_gather(l + 1, p ^ 1)

            @pl.when(l >= 2)
            def _():
                drain_outs(l, p)

            compute(l, p)
            start_outs(l, p)
        return carry

    lax.fori_loop(0, L // 2, step, 0)
    drain_outs(L - 2, 0)
    drain_outs(L - 1, 1)


VBLK = 4096


def _tc_pad_body(wt_ref, out_ref):
    out_ref[:, 0:DIM] = wt_ref[...].T
    out_ref[:, DIM:WPAD] = jnp.zeros((VBLK, WPAD - DIM), jnp.float32)


def _pad_table(wt):
    v = wt.shape[1]
    grid = (v + VBLK - 1) // VBLK
    return pl.pallas_call(
        _tc_pad_body,
        grid=(grid,),
        in_specs=[pl.BlockSpec((DIM, VBLK), lambda i: (0, i))],
        out_specs=pl.BlockSpec((VBLK, WPAD), lambda i: (i, 0)),
        out_shape=jax.ShapeDtypeStruct((v, WPAD), jnp.float32),
    )(wt)


def kernel(input_ids, W, pe):
    mesh = plsc.VectorSubcoreMesh(core_axis_name="c", subcore_axis_name="s")
    out = jax.ShapeDtypeStruct((L * DIM, B), jnp.float32)
    otile = pltpu.VMEM((DIM, RPW), jnp.float32)
    f = functools.partial(
        pl.kernel,
        mesh=mesh,
        out_type=(out, out, out),
        compiler_params=pltpu.CompilerParams(needs_layout_passes=False),
        scratch_types=[
            pltpu.VMEM((DIM, NPE), jnp.float32),
            pltpu.VMEM((L, RPW), jnp.int32),
            pltpu.VMEM((RPW,), jnp.int32),
            pltpu.VMEM((RPW, WPAD), jnp.float32),
            pltpu.VMEM((RPW, WPAD), jnp.float32),
            otile, otile, otile, otile, otile, otile,
            pltpu.SemaphoreType.DMA,
            pltpu.SemaphoreType.DMA,
            pltpu.SemaphoreType.DMA,
            pltpu.SemaphoreType.DMA,
        ],
    )(_sc_embed)
    w_pad = _pad_table(W.T)
    emb, word, pos = f(input_ids.T, w_pad, pe.T)

    def untrans(o):
        return o.reshape(L, DIM, B).transpose(2, 0, 1)

    return untrans(emb), untrans(word), untrans(pos)


# speedup vs baseline: 2.1601x; 1.1163x over previous
import functools

import jax
import jax.numpy as jnp
from jax import lax
from jax.experimental import pallas as pl
from jax.experimental.pallas import tpu as pltpu
from jax.experimental.pallas import tpu_sc as plsc

B = 4096
L = 200
DIM = 64
WPAD = 128
NPE = 256
NW = 32
RPW = B // NW
KV = RPW // 16


def _sc_embed(ids_hbm, w_hbm, pe_hbm, emb_hbm, word_hbm, pos_hbm,
              pe_v, ids_v, nz_v, wrow0, wrow1,
              word_t0, word_t1, pos_t0, pos_t1, emb_t0, emb_t1,
              sem_g0, sem_g1, sem_o0, sem_o1):
    wid = lax.axis_index("s") * 2 + lax.axis_index("c")
    base = wid * RPW

    wrow = (wrow0, wrow1)
    word_t = (word_t0, word_t1)
    pos_t = (pos_t0, pos_t1)
    emb_t = (emb_t0, emb_t1)
    sem_g = (sem_g0, sem_g1)
    sem_o = (sem_o0, sem_o1)
    outs = (word_hbm, pos_hbm, emb_hbm)

    pltpu.sync_copy(pe_hbm.at[:, pl.ds(0, NPE)], pe_v)
    pltpu.sync_copy(ids_hbm.at[:, pl.ds(base, RPW)], ids_v)

    lane = lax.iota(jnp.int32, 16)
    perm = {s: lane ^ s for s in (1, 2, 4, 8)}
    sel = {s: (lane & s) == 0 for s in (1, 2, 4, 8)}

    for k in range(KV):
        def cnt(l, acc):
            v = ids_v[l, pl.ds(k * 16, 16)]
            return acc + jnp.where(v != 0, 1, 0).astype(jnp.int32)
        nz_v[pl.ds(k * 16, 16)] = lax.fori_loop(0, L, cnt, jnp.zeros((16,), jnp.int32))

    def start_gather(l, p):
        pltpu.async_copy(w_hbm.at[ids_v.at[l]], wrow[p], sem_g[p])

    def drain_gather(l, p):
        pltpu.make_async_copy(w_hbm.at[ids_v.at[l]], wrow[p], sem_g[p]).wait()

    def bufs(p):
        return (word_t[p], pos_t[p], emb_t[p])

    def start_outs(l, p):
        for o, t in zip(outs, bufs(p)):
            pltpu.async_copy(t, o.at[pl.ds(l * DIM, DIM), pl.ds(base, RPW)],
                             sem_o[p])

    def drain_outs(l, p):
        for o, t in zip(outs, bufs(p)):
            pltpu.make_async_copy(
                t, o.at[pl.ds(l * DIM, DIM), pl.ds(base, RPW)], sem_o[p]).wait()

    def transpose16(k, d0, p):
        v = [wrow[p][k * 16 + r, pl.ds(d0, 16)] for r in range(16)]
        for s in (1, 2, 4, 8):
            nv = list(v)
            for r in range(16):
                if r & s:
                    continue
                a, b = v[r], v[r ^ s]
                pa = jnp.take(a, perm[s], mode="wrap")
                pb = jnp.take(b, perm[s], mode="wrap")
                nv[r] = jnp.where(sel[s], a, pb)
                nv[r ^ s] = jnp.where(sel[s], pa, b)
            v = nv
        return v

    def compute(l, p):
        def kbody(k, carry):
            nz16 = nz_v[pl.ds(k * 16, 16)]
            pcol = jnp.where((l + 1) <= nz16, l + 1, 0).astype(jnp.int32)

            for db in range(DIM // 16):
                d0 = db * 16
                v = transpose16(k, d0, p)
                for c in range(16):
                    d = d0 + c
                    wvec = v[c]
                    pvec = plsc.load_gather(
                        pe_v, [jnp.full((16,), d, jnp.int32), pcol])
                    word_t[p][d, pl.ds(k * 16, 16)] = wvec
                    pos_t[p][d, pl.ds(k * 16, 16)] = pvec
                    emb_t[p][d, pl.ds(k * 16, 16)] = wvec + pvec
            return carry

        lax.fori_loop(0, KV, kbody, 0)

    start_gather(0, 0)

    def step(lc, carry):
        for p in range(2):
            l = lc * 2 + p
            drain_gather(l, p)

            @pl.when(l + 1 < L)
            def _():
                start_gather(l + 1, p ^ 1)

            @pl.when(l >= 2)
            def _():
                drain_outs(l, p)

            compute(l, p)
            start_outs(l, p)
        return carry

    lax.fori_loop(0, L // 2, step, 0)
    drain_outs(L - 2, 0)
    drain_outs(L - 1, 1)


VBLK = 16384


def _tc_pad_body(wt_ref, out_ref):
    out_ref[:, 0:DIM] = wt_ref[...].T
    out_ref[:, DIM:WPAD] = jnp.zeros((VBLK, WPAD - DIM), jnp.float32)


def _pad_table(wt):
    v = wt.shape[1]
    grid = (v + VBLK - 1) // VBLK
    return pl.pallas_call(
        _tc_pad_body,
        grid=(grid,),
        in_specs=[pl.BlockSpec((DIM, VBLK), lambda i: (0, i))],
        out_specs=pl.BlockSpec((VBLK, WPAD), lambda i: (i, 0)),
        out_shape=jax.ShapeDtypeStruct((v, WPAD), jnp.float32),
    )(wt)


def kernel(input_ids, W, pe):
    mesh = plsc.VectorSubcoreMesh(core_axis_name="c", subcore_axis_name="s")
    out = jax.ShapeDtypeStruct((L * DIM, B), jnp.float32)
    otile = pltpu.VMEM((DIM, RPW), jnp.float32)
    f = functools.partial(
        pl.kernel,
        mesh=mesh,
        out_type=(out, out, out),
        compiler_params=pltpu.CompilerParams(needs_layout_passes=False),
        scratch_types=[
            pltpu.VMEM((DIM, NPE), jnp.float32),
            pltpu.VMEM((L, RPW), jnp.int32),
            pltpu.VMEM((RPW,), jnp.int32),
            pltpu.VMEM((RPW, WPAD), jnp.float32),
            pltpu.VMEM((RPW, WPAD), jnp.float32),
            otile, otile, otile, otile, otile, otile,
            pltpu.SemaphoreType.DMA,
            pltpu.SemaphoreType.DMA,
            pltpu.SemaphoreType.DMA,
            pltpu.SemaphoreType.DMA,
        ],
    )(_sc_embed)
    w_pad = _pad_table(W.T)
    emb, word, pos = f(input_ids.T, w_pad, pe.T)

    def untrans(o):
        return o.reshape(L, DIM, B).transpose(2, 0, 1)

    return untrans(emb), untrans(word), untrans(pos)


# speedup vs baseline: 2.1807x; 1.0096x over previous
import functools

import jax
import jax.numpy as jnp
from jax import lax
from jax.experimental import pallas as pl
from jax.experimental.pallas import tpu as pltpu
from jax.experimental.pallas import tpu_sc as plsc

B = 4096
L = 200
DIM = 64
WPAD = 128
NPE = 256
NW = 32
RPW = B // NW
KV = RPW // 16


def _sc_embed(ids_hbm, w_hbm, pe_hbm, emb_hbm, word_hbm, pos_hbm,
              pe_v, ids_v, nz_v, wrow0, wrow1,
              word_t0, word_t1, pos_t0, pos_t1, emb_t0, emb_t1,
              sem_g0, sem_g1, sem_o0, sem_o1):
    wid = lax.axis_index("s") * 2 + lax.axis_index("c")
    base = wid * RPW

    wrow = (wrow0, wrow1)
    word_t = (word_t0, word_t1)
    pos_t = (pos_t0, pos_t1)
    emb_t = (emb_t0, emb_t1)
    sem_g = (sem_g0, sem_g1)
    sem_o = (sem_o0, sem_o1)
    outs = (word_hbm, pos_hbm, emb_hbm)

    pltpu.sync_copy(pe_hbm.at[:, pl.ds(0, NPE)], pe_v)
    pltpu.sync_copy(ids_hbm.at[:, pl.ds(base, RPW)], ids_v)

    lane = lax.iota(jnp.int32, 16)
    perm = {s: lane ^ s for s in (1, 2, 4, 8)}
    sel = {s: (lane & s) == 0 for s in (1, 2, 4, 8)}

    for k in range(KV):
        def cnt(l, acc):
            v = ids_v[l, pl.ds(k * 16, 16)]
            return acc + jnp.where(v != 0, 1, 0).astype(jnp.int32)
        nz_v[pl.ds(k * 16, 16)] = lax.fori_loop(0, L, cnt, jnp.zeros((16,), jnp.int32))

    def start_gather(l, p):
        pltpu.async_copy(w_hbm.at[ids_v.at[l]], wrow[p], sem_g[p])

    def drain_gather(l, p):
        pltpu.make_async_copy(w_hbm.at[ids_v.at[l]], wrow[p], sem_g[p]).wait()

    def bufs(p):
        return (word_t[p], pos_t[p], emb_t[p])

    def start_outs(l, p):
        for o, t in zip(outs, bufs(p)):
            pltpu.async_copy(t, o.at[pl.ds(l * DIM, DIM), pl.ds(base, RPW)],
                             sem_o[p])

    def drain_outs(l, p):
        for o, t in zip(outs, bufs(p)):
            pltpu.make_async_copy(
                t, o.at[pl.ds(l * DIM, DIM), pl.ds(base, RPW)], sem_o[p]).wait()

    def transpose16(k, d0, p):
        v = [wrow[p][k * 16 + r, pl.ds(d0, 16)] for r in range(16)]
        for s in (1, 2, 4, 8):
            nv = list(v)
            for r in range(16):
                if r & s:
                    continue
                a, b = v[r], v[r ^ s]
                pa = jnp.take(a, perm[s], mode="wrap")
                pb = jnp.take(b, perm[s], mode="wrap")
                nv[r] = jnp.where(sel[s], a, pb)
                nv[r ^ s] = jnp.where(sel[s], pa, b)
            v = nv
        return v

    def compute(l, p):
        def kbody(k, carry):
            nz16 = nz_v[pl.ds(k * 16, 16)]
            pcol = jnp.where((l + 1) <= nz16, l + 1, 0).astype(jnp.int32)

            for db in range(DIM // 16):
                d0 = db * 16
                v = transpose16(k, d0, p)
                for c in range(16):
                    d = d0 + c
                    wvec = v[c]
                    pvec = plsc.load_gather(
                        pe_v, [jnp.full((16,), d, jnp.int32), pcol])
                    word_t[p][d, pl.ds(k * 16, 16)] = wvec
                    pos_t[p][d, pl.ds(k * 16, 16)] = pvec
                    emb_t[p][d, pl.ds(k * 16, 16)] = wvec + pvec
            return carry

        lax.fori_loop(0, KV, kbody, 0)

    start_gather(0, 0)

    def step(lc, carry):
        for p in range(2):
            l = lc * 2 + p
            drain_gather(l, p)

            @pl.when(l + 1 < L)
            def _():
                start_gather(l + 1, p ^ 1)

            @pl.when(l >= 2)
            def _():
                drain_outs(l, p)

            compute(l, p)
            start_outs(l, p)
        return carry

    lax.fori_loop(0, L // 2, step, 0)
    drain_outs(L - 2, 0)
    drain_outs(L - 1, 1)


VBLK = 32768


def _tc_pad_body(wt_ref, out_ref):
    out_ref[:, 0:DIM] = wt_ref[...].T
    out_ref[:, DIM:WPAD] = jnp.zeros((VBLK, WPAD - DIM), jnp.float32)


def _pad_table(wt):
    v = wt.shape[1]
    grid = (v + VBLK - 1) // VBLK
    return pl.pallas_call(
        _tc_pad_body,
        grid=(grid,),
        in_specs=[pl.BlockSpec((DIM, VBLK), lambda i: (0, i))],
        out_specs=pl.BlockSpec((VBLK, WPAD), lambda i: (i, 0)),
        out_shape=jax.ShapeDtypeStruct((v, WPAD), jnp.float32),
    )(wt)


def kernel(input_ids, W, pe):
    mesh = plsc.VectorSubcoreMesh(core_axis_name="c", subcore_axis_name="s")
    out = jax.ShapeDtypeStruct((L * DIM, B), jnp.float32)
    otile = pltpu.VMEM((DIM, RPW), jnp.float32)
    f = functools.partial(
        pl.kernel,
        mesh=mesh,
        out_type=(out, out, out),
        compiler_params=pltpu.CompilerParams(needs_layout_passes=False),
        scratch_types=[
            pltpu.VMEM((DIM, NPE), jnp.float32),
            pltpu.VMEM((L, RPW), jnp.int32),
            pltpu.VMEM((RPW,), jnp.int32),
            pltpu.VMEM((RPW, WPAD), jnp.float32),
            pltpu.VMEM((RPW, WPAD), jnp.float32),
            otile, otile, otile, otile, otile, otile,
            pltpu.SemaphoreType.DMA,
            pltpu.SemaphoreType.DMA,
            pltpu.SemaphoreType.DMA,
            pltpu.SemaphoreType.DMA,
        ],
    )(_sc_embed)
    w_pad = _pad_table(W.T)
    emb, word, pos = f(input_ids.T, w_pad, pe.T)

    def untrans(o):
        return o.reshape(L, DIM, B).transpose(2, 0, 1)

    return untrans(emb), untrans(word), untrans(pos)
